# baseline (speedup 1.0000x reference)
import jax
import jax.numpy as jnp
from jax import lax
from jax.experimental import pallas as pl
from jax.experimental.pallas import tpu as pltpu
from jax.experimental.pallas import tpu_sc as plsc

N_NODES = 10000
N_EDGES = 320000
D_FEAT = 128
ALPHA = 0.1
NITER = 10

NC = 2
NS = 16
NW = NC * NS
EDGES_PER_W = N_EDGES // NW
CHUNK = 50
NCHUNK = EDGES_PER_W // CHUNK
NBUF = 4
ROWS_PER_S = N_NODES // NS
LANES = 16
VPR = D_FEAT // LANES


DCHUNK = 125
DNCHUNK = N_EDGES // NS // DCHUNK


def _deg_body(idx_hbm, out_hbm, cnt_sh, idx_vm, ones_vm, zero_vm, sem):
    c = lax.axis_index("c")
    s = lax.axis_index("s")
    one = jnp.full((LANES,), 1.0, jnp.float32)
    z = jnp.zeros((LANES,), jnp.float32)

    def fill(r, carry):
        ones_vm[r, :] = one
        zero_vm[r, :] = z
        return carry

    lax.fori_loop(0, DCHUNK, fill, 0)
    for t in range(ROWS_PER_S // DCHUNK):
        pltpu.sync_copy(zero_vm, cnt_sh.at[pl.ds(s * ROWS_PER_S + t * DCHUNK, DCHUNK)])
    pltpu.sync_copy(idx_hbm.at[c, s], idx_vm)
    plsc.subcore_barrier()

    def grp(g, carry):
        for b in range(8):
            pltpu.async_copy(ones_vm, cnt_sh.at[idx_vm.at[g * 8 + b]], sem,
                             add=True)
        for b in range(8):
            pltpu.make_async_copy(ones_vm, cnt_sh.at[idx_vm.at[0]], sem).wait()
        return carry

    lax.fori_loop(0, DNCHUNK // 8, grp, 0)
    plsc.subcore_barrier()
    pltpu.sync_copy(cnt_sh.at[pl.ds(s * ROWS_PER_S, ROWS_PER_S)],
                    out_hbm.at[c, pl.ds(s * ROWS_PER_S, ROWS_PER_S)])


_deg = pl.kernel(
    _deg_body,
    out_type=jax.ShapeDtypeStruct((NC, N_NODES, LANES), jnp.float32),
    mesh=plsc.VectorSubcoreMesh(core_axis_name="c", subcore_axis_name="s"),
    scratch_types=[
        pltpu.VMEM_SHARED((N_NODES, LANES), jnp.float32),
        pltpu.VMEM((DNCHUNK, DCHUNK), jnp.int32),
        pltpu.VMEM((DCHUNK, LANES), jnp.float32),
        pltpu.VMEM((DCHUNK, LANES), jnp.float32),
        pltpu.SemaphoreType.DMA,
    ],
    compiler_params=pltpu.CompilerParams(
        use_tc_tiling_on_sc=False, needs_layout_passes=False),
)


def _spmm_body(preds_hbm, row_hbm, col_hbm, out_hbm,
               acc_sh, row_vm, col_vm,
               buf0, buf1, buf2, buf3,
               sg0, sg1, sg2, sg3, ss0, ss1, ss2, ss3):
    c = lax.axis_index("c")
    s = lax.axis_index("s")
    wid = c * NS + s
    bufs = (buf0, buf1, buf2, buf3)
    semg = (sg0, sg1, sg2, sg3)
    sems = (ss0, ss1, ss2, ss3)

    z = jnp.zeros((LANES,), jnp.float32)

    def zrow(r, carry):
        for j in range(VPR):
            buf0[r, pl.ds(j * LANES, LANES)] = z
        return carry

    lax.fori_loop(0, CHUNK, zrow, 0)
    for t in range(ROWS_PER_S // CHUNK):
        pltpu.sync_copy(buf0, acc_sh.at[pl.ds(s * ROWS_PER_S + t * CHUNK, CHUNK)])
    _REM = ROWS_PER_S % CHUNK
    if _REM:
        pltpu.sync_copy(
            buf0.at[pl.ds(0, _REM)],
            acc_sh.at[pl.ds(s * ROWS_PER_S + (ROWS_PER_S // CHUNK) * CHUNK, _REM)])

    pltpu.sync_copy(row_hbm.at[wid], row_vm)
    pltpu.sync_copy(col_hbm.at[wid], col_vm)

    plsc.subcore_barrier()

    for b in range(NBUF):
        pltpu.async_copy(preds_hbm.at[col_vm.at[b]], bufs[b], semg[b])

    def do_chunk(b, i, issue_next):
        pltpu.make_async_copy(preds_hbm.at[col_vm.at[0]], bufs[b], semg[b]).wait()
        pltpu.async_copy(bufs[b], acc_sh.at[row_vm.at[i]], sems[b], add=True)
        pltpu.make_async_copy(bufs[b], acc_sh.at[row_vm.at[0]], sems[b]).wait()
        if issue_next:
            pltpu.async_copy(preds_hbm.at[col_vm.at[i + NBUF]], bufs[b], semg[b])

    def grp(g, carry):
        for b in range(NBUF):
            do_chunk(b, g * NBUF + b, True)
        return carry

    lax.fori_loop(0, NCHUNK // NBUF - 1, grp, 0)
    for b in range(NBUF):
        do_chunk(b, NCHUNK - NBUF + b, False)

    plsc.subcore_barrier()

    pltpu.sync_copy(acc_sh.at[pl.ds(s * ROWS_PER_S, ROWS_PER_S)],
                    out_hbm.at[c, pl.ds(s * ROWS_PER_S, ROWS_PER_S)])


_spmm = pl.kernel(
    _spmm_body,
    out_type=jax.ShapeDtypeStruct((NC, N_NODES, D_FEAT), jnp.float32),
    mesh=plsc.VectorSubcoreMesh(core_axis_name="c", subcore_axis_name="s"),
    scratch_types=[
        pltpu.VMEM_SHARED((N_NODES, D_FEAT), jnp.float32),
        pltpu.VMEM((NCHUNK, CHUNK), jnp.int32),
        pltpu.VMEM((NCHUNK, CHUNK), jnp.int32),
    ] + [pltpu.VMEM((CHUNK, D_FEAT), jnp.float32)] * NBUF
      + [pltpu.SemaphoreType.DMA] * (2 * NBUF),
    compiler_params=pltpu.CompilerParams(
        use_tc_tiling_on_sc=False, needs_layout_passes=False),
)


def kernel(E, edge_index, A_vals):
    row = edge_index[0].astype(jnp.int32)
    col = edge_index[1].astype(jnp.int32)
    idx2 = jnp.stack([row, col]).reshape(NC, NS, DNCHUNK, DCHUNK)
    cnts = _deg(idx2)
    deg_r = jnp.clip(cnts[0, :, 0], 1.0, None)
    deg_c = jnp.clip(cnts[1, :, 0], 1.0, None)
    f = lax.rsqrt(deg_r)[:, None]
    g = (1.0 - ALPHA) * lax.rsqrt(deg_c)[:, None]

    row3 = row.reshape(NW, NCHUNK, CHUNK)
    col3 = col.reshape(NW, NCHUNK, CHUNK)

    gf = g * f
    Q = g * E
    aGE = ALPHA * Q
    for _ in range(NITER - 1):
        S = _spmm(Q, row3, col3)
        Q = gf * (S[0] + S[1]) + aGE
    S = _spmm(Q, row3, col3)
    return f * (S[0] + S[1]) + ALPHA * E


# speedup vs baseline: 1.1080x; 1.1080x over previous
import jax
import jax.numpy as jnp
from jax import lax
from jax.experimental import pallas as pl
from jax.experimental.pallas import tpu as pltpu
from jax.experimental.pallas import tpu_sc as plsc

N_NODES = 10000
N_EDGES = 320000
D_FEAT = 128
ALPHA = 0.1
NITER = 10

NC = 2
NS = 16
D2 = D_FEAT // NC
EPT = N_EDGES // NS
CHUNK = 125
NCHUNK = EPT // CHUNK
NBUF = 4
ROWS_PER_S = N_NODES // NS
BLK = 125
NBLK = ROWS_PER_S // BLK
LANES = 16
VPR = D2 // LANES

DCHUNK = 125
DNCHUNK = N_EDGES // NS // DCHUNK


def _deg_body(idx_hbm, out_hbm, cnt_sh, idx_vm, ones_vm, zero_vm, sem):
    c = lax.axis_index("c")
    s = lax.axis_index("s")
    one = jnp.full((LANES,), 1.0, jnp.float32)
    z = jnp.zeros((LANES,), jnp.float32)

    def fill(r, carry):
        ones_vm[r, :] = one
        zero_vm[r, :] = z
        return carry

    lax.fori_loop(0, DCHUNK, fill, 0)
    for t in range(ROWS_PER_S // DCHUNK):
        pltpu.sync_copy(zero_vm,
                        cnt_sh.at[pl.ds(s * ROWS_PER_S + t * DCHUNK, DCHUNK)])
    pltpu.sync_copy(idx_hbm.at[c, s], idx_vm)
    plsc.subcore_barrier()

    def grp(g, carry):
        for b in range(8):
            pltpu.async_copy(ones_vm, cnt_sh.at[idx_vm.at[g * 8 + b]], sem,
                             add=True)
        for b in range(8):
            pltpu.make_async_copy(ones_vm, cnt_sh.at[idx_vm.at[0]], sem).wait()
        return carry

    lax.fori_loop(0, DNCHUNK // 8, grp, 0)
    plsc.subcore_barrier()
    pltpu.sync_copy(cnt_sh.at[pl.ds(s * ROWS_PER_S, ROWS_PER_S)],
                    out_hbm.at[c, pl.ds(s * ROWS_PER_S, ROWS_PER_S)])


_deg = pl.kernel(
    _deg_body,
    out_type=jax.ShapeDtypeStruct((NC, N_NODES, LANES), jnp.float32),
    mesh=plsc.VectorSubcoreMesh(core_axis_name="c", subcore_axis_name="s"),
    scratch_types=[
        pltpu.VMEM_SHARED((N_NODES, LANES), jnp.float32),
        pltpu.VMEM((DNCHUNK, DCHUNK), jnp.int32),
        pltpu.VMEM((DCHUNK, LANES), jnp.float32),
        pltpu.VMEM((DCHUNK, LANES), jnp.float32),
        pltpu.SemaphoreType.DMA,
    ],
    compiler_params=pltpu.CompilerParams(
        use_tc_tiling_on_sc=False, needs_layout_passes=False),
)


def _power_body(q0_hbm, c_hbm, gf_hbm, f_hbm, row_hbm, col_hbm, out_hbm,
                acc_sh, row_vm, col_vm, buf0, buf1, buf2, buf3, svec_vm,
                sg0, sg1, sg2, sg3, ss0, ss1, ss2, ss3, sr):
    c = lax.axis_index("c")
    s = lax.axis_index("s")
    bufs = (buf0, buf1, buf2, buf3)
    semg = (sg0, sg1, sg2, sg3)
    sems = (ss0, ss1, ss2, ss3)
    qout = out_hbm.at[c]
    cc = c_hbm.at[c]

    pltpu.sync_copy(row_hbm.at[s], row_vm)
    pltpu.sync_copy(col_hbm.at[s], col_vm)
    pltpu.sync_copy(cc.at[pl.ds(s * ROWS_PER_S, ROWS_PER_S)],
                    acc_sh.at[pl.ds(s * ROWS_PER_S, ROWS_PER_S)])
    plsc.subcore_barrier()

    def scatter_phase(qref):
        def do_chunk(b, i, issue_next):
            pltpu.make_async_copy(qref.at[col_vm.at[0]], bufs[b],
                                  semg[b]).wait()
            pltpu.async_copy(bufs[b], acc_sh.at[row_vm.at[i]], sems[b],
                             add=True)
            pltpu.make_async_copy(bufs[b], acc_sh.at[row_vm.at[0]],
                                  sems[b]).wait()
            if issue_next:
                pltpu.async_copy(qref.at[col_vm.at[i + NBUF]], bufs[b],
                                 semg[b])

        for b in range(NBUF):
            pltpu.async_copy(qref.at[col_vm.at[b]], bufs[b], semg[b])

        def grp(gi, carry):
            for b in range(NBUF):
                do_chunk(b, gi * NBUF + b, True)
            return carry

        lax.fori_loop(0, NCHUNK // NBUF - 1, grp, 0)
        for b in range(NBUF):
            do_chunk(b, NCHUNK - NBUF + b, False)

    def scale_phase(svec2d):
        def blk_read(t):
            r0 = s * ROWS_PER_S + t * BLK
            pltpu.async_copy(acc_sh.at[pl.ds(r0, BLK)], bufs[t % NBUF],
                             semg[t % NBUF])

        blk_read(0)
        blk_read(1)
        for t in range(NBLK):
            b = t % NBUF
            r0 = s * ROWS_PER_S + t * BLK
            pltpu.make_async_copy(acc_sh.at[pl.ds(0, BLK)], bufs[b],
                                  semg[b]).wait()
            pltpu.sync_copy(svec2d.at[s * NBLK + t], svec_vm)
            buf = bufs[b]

            def rowloop(r, carry):
                sp = plsc.load_gather(
                    svec_vm, [jnp.full((LANES,), r, jnp.int32)])
                for j in range(VPR):
                    sl = pl.ds(j * LANES, LANES)
                    buf[r, sl] = buf[r, sl] * sp
                return carry

            lax.fori_loop(0, BLK, rowloop, 0)
            pltpu.async_copy(bufs[b], qout.at[pl.ds(r0, BLK)], sems[b])
            pltpu.async_copy(cc.at[pl.ds(r0, BLK)], acc_sh.at[pl.ds(r0, BLK)],
                             sr)
            if t + 2 < NBLK:
                bb = (t + 2) % NBUF
                if t + 2 >= NBUF:
                    pltpu.make_async_copy(bufs[bb], qout.at[pl.ds(0, BLK)],
                                          sems[bb]).wait()
                blk_read(t + 2)
        for t in range(max(0, NBLK - NBUF), NBLK):
            pltpu.make_async_copy(bufs[t % NBUF], qout.at[pl.ds(0, BLK)],
                                  sems[t % NBUF]).wait()
        for t in range(NBLK):
            pltpu.make_async_copy(cc.at[pl.ds(0, BLK)],
                                  acc_sh.at[pl.ds(0, BLK)], sr).wait()

    scatter_phase(q0_hbm.at[c])
    plsc.subcore_barrier()
    scale_phase(gf_hbm)
    plsc.subcore_barrier()

    def it(k, carry):
        scatter_phase(qout)
        plsc.subcore_barrier()

        @pl.when(k < NITER - 2)
        def _():
            scale_phase(gf_hbm)

        @pl.when(k == NITER - 2)
        def _():
            scale_phase(f_hbm)

        plsc.subcore_barrier()
        return carry

    lax.fori_loop(0, NITER - 1, it, 0)


_power = pl.kernel(
    _power_body,
    out_type=jax.ShapeDtypeStruct((NC, N_NODES, D2), jnp.float32),
    mesh=plsc.VectorSubcoreMesh(core_axis_name="c", subcore_axis_name="s"),
    scratch_types=[
        pltpu.VMEM_SHARED((N_NODES, D2), jnp.float32),
        pltpu.VMEM((NCHUNK, CHUNK), jnp.int32),
        pltpu.VMEM((NCHUNK, CHUNK), jnp.int32),
    ] + [pltpu.VMEM((CHUNK, D2), jnp.float32)] * NBUF
      + [pltpu.VMEM((BLK,), jnp.float32)]
      + [pltpu.SemaphoreType.DMA] * (2 * NBUF + 1),
    compiler_params=pltpu.CompilerParams(
        use_tc_tiling_on_sc=False, needs_layout_passes=False),
)


def kernel(E, edge_index, A_vals):
    row = edge_index[0].astype(jnp.int32)
    col = edge_index[1].astype(jnp.int32)
    idx2 = jnp.stack([row, col]).reshape(NC, NS, DNCHUNK, DCHUNK)
    cnts = _deg(idx2)
    deg_r = jnp.clip(cnts[0, :, 0], 1.0, None)
    deg_c = jnp.clip(cnts[1, :, 0], 1.0, None)
    f = lax.rsqrt(deg_r)[:, None]
    g = (1.0 - ALPHA) * lax.rsqrt(deg_c)[:, None]

    row2 = row.reshape(NS, NCHUNK, CHUNK)
    col2 = col.reshape(NS, NCHUNK, CHUNK)

    q0f = g * E
    q0 = jnp.stack([q0f[:, :D2], q0f[:, D2:]])
    cfull = (ALPHA / f) * E
    cmat = jnp.stack([cfull[:, :D2], cfull[:, D2:]])
    gf2 = (g[:, 0] * f[:, 0]).reshape(NS * NBLK, BLK)
    f2 = f[:, 0].reshape(NS * NBLK, BLK)

    out = _power(q0, cmat, gf2, f2, row2, col2)
    return jnp.concatenate([out[0], out[1]], axis=1)
